# baseline (speedup 1.0000x reference)
import functools

import jax
import jax.numpy as jnp
from jax import lax
from jax.experimental import pallas as pl
from jax.experimental.pallas import tpu as pltpu
from jax.experimental.pallas import tpu_sc as plsc

C, CR, K = 1024, 64, 32
CRP = 512
RB = 1024
NW = 32


def _se_topk_body(x_ref, w1p_ref, w2t_ref, wk_ref, fid_ref, keep_ref):
    x = x_ref[...]
    hp = lax.dot_general(x, w1p_ref[...], (((1,), (1,)), ((), ())),
                         preferred_element_type=jnp.float32)
    keep_ref[...] = jnp.max(hp, axis=0, keepdims=True)[None]
    h = jnp.maximum(hp[:, :CR], 0.0)
    z = jnp.dot(h, w2t_ref[...], preferred_element_type=jnp.float32)
    w = jax.nn.sigmoid(z)

    iota_c = jax.lax.broadcasted_iota(jnp.int32, (RB, C), 1)
    iota_k = jax.lax.broadcasted_iota(jnp.int32, (RB, K), 1)

    wk_first = w[:, :K]
    rank_cols = []
    for c in range(K):
        wc = w[:, c:c + 1]
        gt = jnp.sum(jnp.where(w > wc, 1.0, 0.0), axis=-1, keepdims=True)
        eq = jnp.sum(jnp.where((wk_first == wc) & (iota_k < c), 1.0, 0.0),
                     axis=-1, keepdims=True)
        rank_cols.append(gt + eq)
    ranks_f = jnp.concatenate(rank_cols, axis=1)

    pos_cols = []
    for c in range(K):
        rc = ranks_f[:, c:c + 1]
        pos_cols.append(
            jnp.sum(jnp.where(ranks_f < rc, 1.0, 0.0), axis=-1, keepdims=True))
    pos = jnp.concatenate(pos_cols, axis=1).astype(jnp.int32)

    col_cols = []
    for m in range(K):
        col_cols.append(
            jnp.sum(jnp.where(pos == m, ranks_f, 0.0), axis=-1, keepdims=True))
    cols = jnp.concatenate(col_cols, axis=1).astype(jnp.int32)

    pid = pl.program_id(0)
    row = jax.lax.broadcasted_iota(jnp.int32, (RB, K), 0) + pid * RB
    fid_ref[...] = row * C + cols

    wcur = w
    mxs, csums = [], []
    csum = None
    for s in range(K):
        mx = jnp.max(wcur, axis=-1, keepdims=True)
        ismax = wcur == mx
        cnt = jnp.sum(jnp.where(ismax, 1.0, 0.0), axis=-1, keepdims=True)
        wcur = jnp.where(ismax, -1.0, wcur)
        mxs.append(mx)
        csum = cnt if csum is None else csum + cnt
        csums.append(csum)
    iota_kf = iota_k.astype(jnp.float32)
    wk_acc = jnp.zeros((RB, K), jnp.float32)
    prev = jnp.zeros((RB, 1), jnp.float32)
    for s in range(K):
        sel = (iota_kf >= prev) & (iota_kf < csums[s])
        wk_acc = wk_acc + jnp.where(sel, mxs[s], 0.0)
        prev = csums[s]
    wk_ref[...] = wk_acc


def _make_sc_gather(total, ch):
    mesh = plsc.VectorSubcoreMesh(core_axis_name="c", subcore_axis_name="s")

    @functools.partial(
        pl.kernel, mesh=mesh,
        out_type=jax.ShapeDtypeStruct((total,), jnp.float32),
        scratch_types=[
            pltpu.VMEM((ch,), jnp.int32),
            pltpu.VMEM((ch,), jnp.float32),
            pltpu.VMEM((ch,), jnp.float32),
            pltpu.SemaphoreType.DMA,
        ],
    )
    def sc_gather(xf_hbm, idx_hbm, wk_hbm, out_hbm, idx_v, gat_v, wk_v, sem):
        wid = lax.axis_index("s") * 2 + lax.axis_index("c")
        base = wid * ch
        pltpu.sync_copy(idx_hbm.at[pl.ds(base, ch)], idx_v)
        pltpu.async_copy(xf_hbm.at[idx_v], gat_v, sem).wait()
        pltpu.sync_copy(wk_hbm.at[pl.ds(base, ch)], wk_v)

        def body(i, carry):
            s = pl.ds(i * 16, 16)
            gat_v[s] = gat_v[s] * wk_v[s]
            return carry

        lax.fori_loop(0, ch // 16, body, 0)
        pltpu.sync_copy(gat_v, out_hbm.at[pl.ds(base, ch)])

    return sc_gather


def kernel(x, W1, W2):
    n = x.shape[0]
    w1p = jnp.zeros((CRP, C), jnp.float32).at[:CR].set(W1)
    wk, fid, _ = pl.pallas_call(
        _se_topk_body,
        grid=(n // RB,),
        in_specs=[
            pl.BlockSpec((RB, C), lambda i: (i, 0)),
            pl.BlockSpec((CRP, C), lambda i: (0, 0)),
            pl.BlockSpec((CR, C), lambda i: (0, 0)),
        ],
        out_specs=[
            pl.BlockSpec((RB, K), lambda i: (i, 0)),
            pl.BlockSpec((RB, K), lambda i: (i, 0)),
            pl.BlockSpec((1, 1, CRP), lambda i: (i, 0, 0)),
        ],
        out_shape=[
            jax.ShapeDtypeStruct((n, K), jnp.float32),
            jax.ShapeDtypeStruct((n, K), jnp.int32),
            jax.ShapeDtypeStruct((n // RB, 1, CRP), jnp.float32),
        ],
    )(x, w1p, W2.T)

    total = n * K
    sc_gather = _make_sc_gather(total, total // NW)
    out_flat = sc_gather(x.reshape(-1), fid.reshape(-1), wk.reshape(-1))
    return out_flat.reshape(n, K)


# speedup vs baseline: 1.1171x; 1.1171x over previous
import functools

import jax
import jax.numpy as jnp
from jax import lax
from jax.experimental import pallas as pl
from jax.experimental.pallas import tpu as pltpu
from jax.experimental.pallas import tpu_sc as plsc

C, CR, K = 1024, 64, 32
CRP = 512
RB = 512
NW = 32


def _se_topk_body(x_ref, w1p_ref, w2t_ref, wk_ref, fid_ref, keep_ref):
    x = x_ref[...]
    hp = lax.dot_general(x, w1p_ref[...], (((1,), (1,)), ((), ())),
                         preferred_element_type=jnp.float32)
    keep_ref[...] = jnp.max(hp, axis=0, keepdims=True)[None]
    h = jnp.maximum(hp[:, :CR], 0.0)
    z = jnp.dot(h, w2t_ref[...], preferred_element_type=jnp.float32)
    w = jax.nn.sigmoid(z)

    iota_c = jax.lax.broadcasted_iota(jnp.int32, (RB, C), 1)
    iota_k = jax.lax.broadcasted_iota(jnp.int32, (RB, K), 1)

    wk_first = w[:, :K]
    rank_cols = []
    for c in range(K):
        wc = w[:, c:c + 1]
        gt = jnp.sum(jnp.where(w > wc, 1.0, 0.0), axis=-1, keepdims=True)
        eq = jnp.sum(jnp.where((wk_first == wc) & (iota_k < c), 1.0, 0.0),
                     axis=-1, keepdims=True)
        rank_cols.append(gt + eq)
    ranks_f = jnp.concatenate(rank_cols, axis=1)

    pos_cols = []
    for c in range(K):
        rc = ranks_f[:, c:c + 1]
        pos_cols.append(
            jnp.sum(jnp.where(ranks_f < rc, 1.0, 0.0), axis=-1, keepdims=True))
    pos = jnp.concatenate(pos_cols, axis=1).astype(jnp.int32)

    col_cols = []
    for m in range(K):
        col_cols.append(
            jnp.sum(jnp.where(pos == m, ranks_f, 0.0), axis=-1, keepdims=True))
    cols = jnp.concatenate(col_cols, axis=1).astype(jnp.int32)

    pid = pl.program_id(0)
    row = jax.lax.broadcasted_iota(jnp.int32, (RB, K), 0) + pid * RB
    fid_ref[...] = row * C + cols

    wcur = w
    mxs, csums = [], []
    csum = None
    for s in range(K):
        mx = jnp.max(wcur, axis=-1, keepdims=True)
        ismax = wcur == mx
        cnt = jnp.sum(jnp.where(ismax, 1.0, 0.0), axis=-1, keepdims=True)
        wcur = jnp.where(ismax, -1.0, wcur)
        mxs.append(mx)
        csum = cnt if csum is None else csum + cnt
        csums.append(csum)
    iota_kf = iota_k.astype(jnp.float32)
    wk_acc = jnp.zeros((RB, K), jnp.float32)
    prev = jnp.zeros((RB, 1), jnp.float32)
    for s in range(K):
        sel = (iota_kf >= prev) & (iota_kf < csums[s])
        wk_acc = wk_acc + jnp.where(sel, mxs[s], 0.0)
        prev = csums[s]
    wk_ref[...] = wk_acc


def _make_sc_gather(total, ch):
    mesh = plsc.VectorSubcoreMesh(core_axis_name="c", subcore_axis_name="s")

    @functools.partial(
        pl.kernel, mesh=mesh,
        out_type=jax.ShapeDtypeStruct((total,), jnp.float32),
        scratch_types=[
            pltpu.VMEM((ch,), jnp.int32),
            pltpu.VMEM((ch,), jnp.float32),
            pltpu.VMEM((ch,), jnp.float32),
            pltpu.SemaphoreType.DMA,
        ],
    )
    def sc_gather(xf_hbm, idx_hbm, wk_hbm, out_hbm, idx_v, gat_v, wk_v, sem):
        wid = lax.axis_index("s") * 2 + lax.axis_index("c")
        base = wid * ch
        pltpu.sync_copy(idx_hbm.at[pl.ds(base, ch)], idx_v)
        pltpu.async_copy(xf_hbm.at[idx_v], gat_v, sem).wait()
        pltpu.sync_copy(wk_hbm.at[pl.ds(base, ch)], wk_v)

        def body(i, carry):
            s = pl.ds(i * 16, 16)
            gat_v[s] = gat_v[s] * wk_v[s]
            return carry

        lax.fori_loop(0, ch // 16, body, 0)
        pltpu.sync_copy(gat_v, out_hbm.at[pl.ds(base, ch)])

    return sc_gather


def kernel(x, W1, W2):
    n = x.shape[0]
    w1p = jnp.zeros((CRP, C), jnp.float32).at[:CR].set(W1)
    wk, fid, _ = pl.pallas_call(
        _se_topk_body,
        grid=(n // RB,),
        in_specs=[
            pl.BlockSpec((RB, C), lambda i: (i, 0)),
            pl.BlockSpec((CRP, C), lambda i: (0, 0)),
            pl.BlockSpec((CR, C), lambda i: (0, 0)),
        ],
        out_specs=[
            pl.BlockSpec((RB, K), lambda i: (i, 0)),
            pl.BlockSpec((RB, K), lambda i: (i, 0)),
            pl.BlockSpec((1, 1, CRP), lambda i: (i, 0, 0)),
        ],
        out_shape=[
            jax.ShapeDtypeStruct((n, K), jnp.float32),
            jax.ShapeDtypeStruct((n, K), jnp.int32),
            jax.ShapeDtypeStruct((n // RB, 1, CRP), jnp.float32),
        ],
    )(x, w1p, W2.T)

    total = n * K
    sc_gather = _make_sc_gather(total, total // NW)
    out_flat = sc_gather(x.reshape(-1), fid.reshape(-1), wk.reshape(-1))
    return out_flat.reshape(n, K)
